# baseline (speedup 1.0000x reference)
import functools

import jax
import jax.numpy as jnp
from jax.experimental import pallas as pl
from jax.experimental.pallas import tpu as pltpu

_RATIOS = (0.1, 0.05)
_INTERPRET = False


def _proj_kernel(x_ref, w_ref, z_ref):
    y = jax.lax.dot_general(x_ref[...], w_ref[...],
                            (((1,), (1,)), ((), ())),
                            preferred_element_type=jnp.float32)
    nrm = jnp.sqrt(jnp.sum(y * y, axis=1, keepdims=True))
    z_ref[...] = y / jnp.maximum(nrm, 1e-12)


def _project(feats, w):
    n, din = feats.shape
    dout = w.shape[0]
    blk = 512
    return pl.pallas_call(
        _proj_kernel,
        grid=(n // blk,),
        in_specs=[
            pl.BlockSpec((blk, din), lambda i: (i, 0)),
            pl.BlockSpec((dout, din), lambda i: (0, 0)),
        ],
        out_specs=pl.BlockSpec((blk, dout), lambda i: (i, 0)),
        out_shape=jax.ShapeDtypeStruct((n, dout), jnp.float32),
        interpret=_INTERPRET,
    )(feats, w)


def _mask_kernel(kp1, zt_ref, zf_ref, m_ref, mt_ref):
    zt = zt_ref[...]
    zf = zf_ref[...]
    g = jax.lax.dot_general(zt, zf, (((1,), (1,)), ((), ())),
                            preferred_element_type=jnp.float32)
    sq_f = jnp.sum(zf * zf, axis=1)[None, :]
    sq_t = jnp.sum(zt * zt, axis=1)[:, None]
    d2 = jnp.maximum(sq_t + sq_f - 2.0 * g, 0.0)
    key = jax.lax.bitcast_convert_type(d2, jnp.int32)
    rows = zt.shape[0]
    lo = jnp.zeros((rows, 1), jnp.int32)
    hi = jnp.full((rows, 1), 0x40810000, jnp.int32)
    ones_col = jnp.ones((d2.shape[1], 8), jnp.bfloat16)

    def body(_, lohi):
        lo, hi = lohi
        mid = lo + (hi - lo) // 2
        midf = jax.lax.bitcast_convert_type(mid, jnp.float32)
        ind = jnp.where(d2 <= midf, 1.0, 0.0).astype(jnp.bfloat16)
        cnt = jax.lax.dot_general(ind, ones_col, (((1,), (0,)), ((), ())),
                                  preferred_element_type=jnp.float32)[:, :1]
        ge = cnt >= jnp.float32(kp1)
        return jnp.where(ge, lo, mid + 1), jnp.where(ge, mid, hi)

    _, thr = jax.lax.fori_loop(0, 31, body, (lo, hi))
    mask = key <= thr
    packed = jnp.zeros((rows, 128), jnp.int32)
    for b in range(32):
        packed = packed | (mask[:, 128 * b:128 * (b + 1)].astype(jnp.int32)
                           << b)
    m_ref[...] = packed
    i = pl.program_id(0)
    tf = mask.astype(jnp.float32).T.astype(jnp.int32)
    contrib = ((tf[:, :128] << (2 * i)) | (tf[:, 128:] << (2 * i + 1)))

    @pl.when(i == 0)
    def _init():
        mt_ref[...] = jnp.zeros_like(mt_ref)

    mt_ref[...] = mt_ref[...] | contrib


def _masks(z, kp1):
    n, d = z.shape
    blk = 256
    return pl.pallas_call(
        functools.partial(_mask_kernel, kp1),
        grid=(n // blk,),
        in_specs=[
            pl.BlockSpec((blk, d), lambda i: (i, 0)),
            pl.BlockSpec((n, d), lambda i: (0, 0)),
        ],
        out_specs=[
            pl.BlockSpec((blk, 128), lambda i: (i, 0)),
            pl.BlockSpec((n, 128), lambda i: (0, 0)),
        ],
        out_shape=[
            jax.ShapeDtypeStruct((n, 128), jnp.int32),
            jax.ShapeDtypeStruct((n, 128), jnp.int32),
        ],
        interpret=_INTERPRET,
    )(z, z)


def _prop_kernel(mp_ref, mtp_ref, out_ref, sym_ref, row_ref, col_ref):
    n = sym_ref.shape[0]
    chunk = 512
    nchunks = n // chunk
    big = jnp.int32(1 << 30)
    symp = mp_ref[...] | mtp_ref[...]
    for b in range(32):
        sym_ref[:, 128 * b:128 * (b + 1)] = \
            (((symp >> b) & 1) ^ 1).astype(jnp.int8)
    row_ref[...] = jax.lax.broadcasted_iota(jnp.int32, (1, n), 1)
    col_ref[...] = jax.lax.broadcasted_iota(jnp.int32, (n, 1), 0)

    def sweep(state):
        del state
        lab_row = row_ref[...]

        def chunk_body(c, carry):
            r2_acc, chg = carry
            pen = sym_ref[pl.ds(c * chunk, chunk), :].astype(jnp.int32) << 30
            lab_col_c = col_ref[pl.ds(c * chunk, chunk), :]
            r1 = jnp.min(lab_row + pen, axis=1, keepdims=True)
            new_col = jnp.minimum(lab_col_c, r1)
            r2_part = jnp.min(lab_col_c + pen, axis=0, keepdims=True)
            col_ref[pl.ds(c * chunk, chunk), :] = new_col
            chg = chg + jnp.sum((new_col != lab_col_c).astype(jnp.int32))
            return jnp.minimum(r2_acc, r2_part), chg

        r2_acc, chg = jax.lax.fori_loop(
            0, nchunks, chunk_body,
            (jnp.full((1, n), big, jnp.int32), jnp.int32(0)))
        row_ref[...] = jnp.minimum(lab_row, r2_acc)
        return chg

    jax.lax.while_loop(lambda chg: chg > 0, sweep, jnp.int32(1))
    out_ref[...] = row_ref[...]


def _components(mp, mtp, n):
    return pl.pallas_call(
        _prop_kernel,
        in_specs=[
            pl.BlockSpec((n, 128), lambda: (0, 0)),
            pl.BlockSpec((n, 128), lambda: (0, 0)),
        ],
        out_specs=pl.BlockSpec((1, n), lambda: (0, 0)),
        out_shape=jax.ShapeDtypeStruct((1, n), jnp.int32),
        scratch_shapes=[
            pltpu.VMEM((n, n), jnp.int8),
            pltpu.VMEM((1, n), jnp.int32),
            pltpu.VMEM((n, 1), jnp.int32),
        ],
        interpret=_INTERPRET,
    )(mp, mtp)


def _finish_kernel(e_minus_n, l0_ref, l1_ref, out_ref):
    n = l0_ref.shape[1]
    iota = jax.lax.broadcasted_iota(jnp.int32, (1, n), 1)
    c0 = jnp.sum((l0_ref[...] == iota).astype(jnp.int32))
    c1 = jnp.sum((l1_ref[...] == iota).astype(jnp.int32))
    b0 = c0 + c1
    b1 = (jnp.maximum(0, e_minus_n[0] + c0) +
          jnp.maximum(0, e_minus_n[1] + c1))
    out_ref[...] = jnp.concatenate(
        [b0.reshape(1, 1), b1.reshape(1, 1)], axis=1).astype(jnp.float32)


def kernel(feats, W0, W1):
    if feats.ndim == 4:
        feats = feats.mean(axis=(2, 3))
    feats = feats.astype(jnp.float32)
    n = feats.shape[0]
    labels = []
    e_minus_n = []
    for i, w in enumerate((W0, W1)):
        k = max(3, int(_RATIOS[i] * n))
        kk = min(k, n - 1)
        z = _project(feats, w)
        mp, mtp = _masks(z, kk + 1)
        labels.append(_components(mp, mtp, n))
        e_minus_n.append(n * kk - n)
    out = pl.pallas_call(
        functools.partial(_finish_kernel, tuple(e_minus_n)),
        in_specs=[
            pl.BlockSpec((1, n), lambda: (0, 0)),
            pl.BlockSpec((1, n), lambda: (0, 0)),
        ],
        out_specs=pl.BlockSpec((1, 2), lambda: (0, 0)),
        out_shape=jax.ShapeDtypeStruct((1, 2), jnp.float32),
        interpret=_INTERPRET,
    )(labels[0], labels[1])
    return out.reshape(2)


# speedup vs baseline: 1.3089x; 1.3089x over previous
import functools

import jax
import jax.numpy as jnp
from jax.experimental import pallas as pl
from jax.experimental.pallas import tpu as pltpu

_RATIOS = (0.1, 0.05)
_INTERPRET = False


def _proj_kernel(x_ref, w_ref, z_ref):
    y = jax.lax.dot_general(x_ref[...], w_ref[...],
                            (((1,), (1,)), ((), ())),
                            preferred_element_type=jnp.float32)
    nrm = jnp.sqrt(jnp.sum(y * y, axis=1, keepdims=True))
    z_ref[...] = y / jnp.maximum(nrm, 1e-12)


def _project(feats, w):
    n, din = feats.shape
    dout = w.shape[0]
    blk = 512
    return pl.pallas_call(
        _proj_kernel,
        grid=(n // blk,),
        in_specs=[
            pl.BlockSpec((blk, din), lambda i: (i, 0)),
            pl.BlockSpec((dout, din), lambda i: (0, 0)),
        ],
        out_specs=pl.BlockSpec((blk, dout), lambda i: (i, 0)),
        out_shape=jax.ShapeDtypeStruct((n, dout), jnp.float32),
        interpret=_INTERPRET,
    )(feats, w)


def _mask_kernel(kp1, zt_ref, zf_ref, m_ref, mt_ref):
    zt = zt_ref[...]
    zf = zf_ref[...]
    g = jax.lax.dot_general(zt, zf, (((1,), (1,)), ((), ())),
                            preferred_element_type=jnp.float32)
    sq_f = jnp.sum(zf * zf, axis=1)[None, :]
    sq_t = jnp.sum(zt * zt, axis=1)[:, None]
    d2 = jnp.maximum(sq_t + sq_f - 2.0 * g, 0.0)
    rows = zt.shape[0]
    lo = jnp.zeros((rows, 1), jnp.int32)
    hi = jnp.full((rows, 1), 0x40810000, jnp.int32)

    def body(_, lohi):
        lo, hi = lohi
        mid = lo + (hi - lo) // 2
        midf = jax.lax.bitcast_convert_type(mid, jnp.float32)
        cnt = jnp.sum(jnp.where(d2 <= midf, 1.0, 0.0), axis=1,
                      keepdims=True)
        ge = cnt >= jnp.float32(kp1)
        return jnp.where(ge, lo, mid + 1), jnp.where(ge, mid, hi)

    _, thr = jax.lax.fori_loop(0, 31, body, (lo, hi))
    thrf = jax.lax.bitcast_convert_type(thr, jnp.float32)
    maskf = jnp.where(d2 <= thrf, 1.0, 0.0)
    m32 = maskf.astype(jnp.int32)
    packed = jnp.zeros((rows, 128), jnp.int32)
    for b in range(32):
        packed = packed | (m32[:, 128 * b:128 * (b + 1)] << b)
    m_ref[...] = packed
    i = pl.program_id(0)
    tf = maskf.T.astype(jnp.int32)
    nb = rows // 128
    contrib = tf[:, :128] << (nb * i)
    for b in range(1, nb):
        contrib = contrib | (tf[:, 128 * b:128 * (b + 1)] << (nb * i + b))

    @pl.when(i == 0)
    def _init():
        mt_ref[...] = jnp.zeros_like(mt_ref)

    mt_ref[...] = mt_ref[...] | contrib


def _masks(z, kp1):
    n, d = z.shape
    blk = 512
    return pl.pallas_call(
        functools.partial(_mask_kernel, kp1),
        grid=(n // blk,),
        in_specs=[
            pl.BlockSpec((blk, d), lambda i: (i, 0)),
            pl.BlockSpec((n, d), lambda i: (0, 0)),
        ],
        out_specs=[
            pl.BlockSpec((blk, 128), lambda i: (i, 0)),
            pl.BlockSpec((n, 128), lambda i: (0, 0)),
        ],
        out_shape=[
            jax.ShapeDtypeStruct((n, 128), jnp.int32),
            jax.ShapeDtypeStruct((n, 128), jnp.int32),
        ],
        interpret=_INTERPRET,
    )(z, z)


def _prop_kernel(mp_ref, mtp_ref, out_ref, sym_ref, row_ref, col_ref):
    n = sym_ref.shape[0]
    chunk = 512
    nchunks = n // chunk
    big = jnp.int32(1 << 30)
    symp = mp_ref[...] | mtp_ref[...]
    for b in range(32):
        sym_ref[:, 128 * b:128 * (b + 1)] = \
            (((symp >> b) & 1) ^ 1).astype(jnp.int8)
    row_ref[...] = jax.lax.broadcasted_iota(jnp.int32, (1, n), 1)
    col_ref[...] = jax.lax.broadcasted_iota(jnp.int32, (n, 1), 0)

    def sweep(state):
        del state
        lab_row = row_ref[...]

        def chunk_body(c, carry):
            r2_acc, chg = carry
            pen = sym_ref[pl.ds(c * chunk, chunk), :].astype(jnp.int32) << 30
            lab_col_c = col_ref[pl.ds(c * chunk, chunk), :]
            r1 = jnp.min(lab_row + pen, axis=1, keepdims=True)
            new_col = jnp.minimum(lab_col_c, r1)
            r2_part = jnp.min(lab_col_c + pen, axis=0, keepdims=True)
            col_ref[pl.ds(c * chunk, chunk), :] = new_col
            chg = chg + jnp.sum((new_col != lab_col_c).astype(jnp.int32))
            return jnp.minimum(r2_acc, r2_part), chg

        r2_acc, chg = jax.lax.fori_loop(
            0, nchunks, chunk_body,
            (jnp.full((1, n), big, jnp.int32), jnp.int32(0)))
        row_ref[...] = jnp.minimum(lab_row, r2_acc)
        return chg

    jax.lax.while_loop(lambda chg: chg > 0, sweep, jnp.int32(1))
    out_ref[...] = row_ref[...]


def _components(mp, mtp, n):
    return pl.pallas_call(
        _prop_kernel,
        in_specs=[
            pl.BlockSpec((n, 128), lambda: (0, 0)),
            pl.BlockSpec((n, 128), lambda: (0, 0)),
        ],
        out_specs=pl.BlockSpec((1, n), lambda: (0, 0)),
        out_shape=jax.ShapeDtypeStruct((1, n), jnp.int32),
        scratch_shapes=[
            pltpu.VMEM((n, n), jnp.int8),
            pltpu.VMEM((1, n), jnp.int32),
            pltpu.VMEM((n, 1), jnp.int32),
        ],
        interpret=_INTERPRET,
    )(mp, mtp)


def _finish_kernel(e_minus_n, l0_ref, l1_ref, out_ref):
    n = l0_ref.shape[1]
    iota = jax.lax.broadcasted_iota(jnp.int32, (1, n), 1)
    c0 = jnp.sum((l0_ref[...] == iota).astype(jnp.int32))
    c1 = jnp.sum((l1_ref[...] == iota).astype(jnp.int32))
    b0 = c0 + c1
    b1 = (jnp.maximum(0, e_minus_n[0] + c0) +
          jnp.maximum(0, e_minus_n[1] + c1))
    out_ref[...] = jnp.concatenate(
        [b0.reshape(1, 1), b1.reshape(1, 1)], axis=1).astype(jnp.float32)


def kernel(feats, W0, W1):
    if feats.ndim == 4:
        feats = feats.mean(axis=(2, 3))
    feats = feats.astype(jnp.float32)
    n = feats.shape[0]
    labels = []
    e_minus_n = []
    for i, w in enumerate((W0, W1)):
        k = max(3, int(_RATIOS[i] * n))
        kk = min(k, n - 1)
        z = _project(feats, w)
        mp, mtp = _masks(z, kk + 1)
        labels.append(_components(mp, mtp, n))
        e_minus_n.append(n * kk - n)
    out = pl.pallas_call(
        functools.partial(_finish_kernel, tuple(e_minus_n)),
        in_specs=[
            pl.BlockSpec((1, n), lambda: (0, 0)),
            pl.BlockSpec((1, n), lambda: (0, 0)),
        ],
        out_specs=pl.BlockSpec((1, 2), lambda: (0, 0)),
        out_shape=jax.ShapeDtypeStruct((1, 2), jnp.float32),
        interpret=_INTERPRET,
    )(labels[0], labels[1])
    return out.reshape(2)
